# Initial kernel scaffold; baseline (speedup 1.0000x reference)
import jax
import jax.numpy as jnp
from jax.experimental import pallas as pl


def kernel(inputs, w):
    raise NotImplementedError("write your pallas kernel here")


# speedup vs baseline: 3.6937x; 3.6937x over previous
import jax
import jax.numpy as jnp
from jax.experimental import pallas as pl
from jax.experimental.pallas import tpu as pltpu

_DIM = 32
_NE = 8192
_N = 8192
_BR1, _BC1 = 256, 1024
_BR2, _BC2 = 256, 2048


def _dist_body(x_ref, w_ref, d_ref, idx_ref, minv, mini):
    c = pl.program_id(1)
    nc = pl.num_programs(1)
    x = x_ref[...]
    wt = w_ref[...]
    x2 = jnp.sum(x * x, axis=1, keepdims=True)
    w2 = jnp.sum(wt * wt, axis=1)[None, :]
    mm = jax.lax.dot_general(x, wt, (((1,), (1,)), ((), ())),
                             preferred_element_type=jnp.float32)
    d = (x2 - 2.0 * mm) + w2
    d_ref[...] = d
    rmin = jnp.min(d, axis=1, keepdims=True)
    col = jax.lax.broadcasted_iota(jnp.int32, d.shape, 1) + c * _BC1
    rarg = jnp.min(jnp.where(d == rmin, col, jnp.int32(2**30)),
                   axis=1, keepdims=True)

    @pl.when(c == 0)
    def _():
        minv[...] = jnp.full_like(minv[...], jnp.inf)
        mini[...] = jnp.zeros_like(mini[...])

    upd = rmin < minv[...]
    mini[...] = jnp.where(upd, rarg, mini[...])
    minv[...] = jnp.where(upd, rmin, minv[...])

    @pl.when(c == nc - 1)
    def _():
        idx_ref[...] = mini[...]


def _enc_body(idx_ref, w_ref, x_ref, enc_ref, q_ref, loss_ref, perp_ref,
              hist, sse, qacc):
    r = pl.program_id(0)
    nr = pl.num_programs(0)
    c = pl.program_id(1)
    nc = pl.num_programs(1)
    idx = idx_ref[...]
    col = jax.lax.broadcasted_iota(jnp.int32, (_BR2, _BC2), 1) + c * _BC2
    enc = (col == idx).astype(jnp.float32)
    enc_ref[...] = enc
    colsum = jnp.sum(enc, axis=0, keepdims=True)

    @pl.when(r == 0)
    def _():
        hist[:, pl.ds(c * _BC2, _BC2)] = colsum

    @pl.when(r > 0)
    def _():
        hist[:, pl.ds(c * _BC2, _BC2)] += colsum

    part = jax.lax.dot_general(enc, w_ref[...], (((1,), (0,)), ((), ())),
                               preferred_element_type=jnp.float32)

    @pl.when(c == 0)
    def _():
        qacc[...] = part

    @pl.when(c > 0)
    def _():
        qacc[...] += part

    @pl.when(c == nc - 1)
    def _():
        xt = x_ref[...]
        diff = qacc[...] - xt
        q_ref[...] = xt + diff
        tile_sse = jnp.sum(diff * diff)
        prev = jnp.where(r == 0, 0.0, sse[0, 0])
        sse[0, 0] = prev + tile_sse

    @pl.when((r == nr - 1) & (c == nc - 1))
    def _():
        loss_ref[0, 0] = sse[0, 0] * (1.25 / float(_N * _DIM))
        avg = hist[...] * (1.0 / float(_N))
        ent = jnp.sum(avg * jnp.log(avg + 1e-10))
        perp_ref[0, 0] = jnp.exp(-ent)


def kernel(inputs, w):
    x = jnp.transpose(inputs, (0, 2, 3, 1))
    input_shape = x.shape
    flat = x.reshape(_N, _DIM)

    dist, idx = pl.pallas_call(
        _dist_body,
        grid=(_N // _BR1, _NE // _BC1),
        in_specs=[
            pl.BlockSpec((_BR1, _DIM), lambda r, c: (r, 0)),
            pl.BlockSpec((_BC1, _DIM), lambda r, c: (c, 0)),
        ],
        out_specs=[
            pl.BlockSpec((_BR1, _BC1), lambda r, c: (r, c)),
            pl.BlockSpec((_BR1, 1), lambda r, c: (r, 0)),
        ],
        out_shape=[
            jax.ShapeDtypeStruct((_N, _NE), jnp.float32),
            jax.ShapeDtypeStruct((_N, 1), jnp.int32),
        ],
        scratch_shapes=[
            pltpu.VMEM((_BR1, 1), jnp.float32),
            pltpu.VMEM((_BR1, 1), jnp.int32),
        ],
    )(flat, w)

    enc, q, loss, perp = pl.pallas_call(
        _enc_body,
        grid=(_N // _BR2, _NE // _BC2),
        in_specs=[
            pl.BlockSpec((_BR2, 1), lambda r, c: (r, 0)),
            pl.BlockSpec((_BC2, _DIM), lambda r, c: (c, 0)),
            pl.BlockSpec((_BR2, _DIM), lambda r, c: (r, 0)),
        ],
        out_specs=[
            pl.BlockSpec((_BR2, _BC2), lambda r, c: (r, c)),
            pl.BlockSpec((_BR2, _DIM), lambda r, c: (r, 0)),
            pl.BlockSpec((1, 1), lambda r, c: (0, 0),
                         memory_space=pltpu.SMEM),
            pl.BlockSpec((1, 1), lambda r, c: (0, 0),
                         memory_space=pltpu.SMEM),
        ],
        out_shape=[
            jax.ShapeDtypeStruct((_N, _NE), jnp.float32),
            jax.ShapeDtypeStruct((_N, _DIM), jnp.float32),
            jax.ShapeDtypeStruct((1, 1), jnp.float32),
            jax.ShapeDtypeStruct((1, 1), jnp.float32),
        ],
        scratch_shapes=[
            pltpu.VMEM((1, _NE), jnp.float32),
            pltpu.SMEM((1, 1), jnp.float32),
            pltpu.VMEM((_BR2, _DIM), jnp.float32),
        ],
    )(idx, w, flat)

    quantized = jnp.transpose(q.reshape(input_shape), (0, 3, 1, 2))
    return (dist, quantized, loss[0, 0], enc, idx, perp[0, 0])


# speedup vs baseline: 3.7367x; 1.0117x over previous
import jax
import jax.numpy as jnp
from jax.experimental import pallas as pl
from jax.experimental.pallas import tpu as pltpu

_DIM = 32
_NE = 8192
_N = 8192
_BR1, _BC1 = 256, 1024
_BR2, _BC2 = 256, 2048


def _dist_body(x_ref, w_ref, d_ref, idx_ref, minv, mini, xsq, w2c, w2x):
    r = pl.program_id(0)
    c = pl.program_id(1)
    nc = pl.num_programs(1)
    x = x_ref[...]

    @pl.when(c == 0)
    def _():
        xsq[...] = jnp.sum(x * x, axis=1, keepdims=True)

    @pl.when(r == 0)
    def _():
        wt = w_ref[...]
        w2x[pl.ds(c * _BC1, _BC1), :] = wt + wt
        w2c[:, pl.ds(c * _BC1, _BC1)] = jnp.sum(wt * wt, axis=1)[None, :]

    x2 = xsq[...]
    w2 = w2c[:, pl.ds(c * _BC1, _BC1)]
    mm2 = jax.lax.dot_general(x, w2x[pl.ds(c * _BC1, _BC1), :],
                              (((1,), (1,)), ((), ())),
                              preferred_element_type=jnp.float32)
    d = (x2 - mm2) + w2
    d_ref[...] = d
    rmin = jnp.min(d, axis=1, keepdims=True)
    col = jax.lax.broadcasted_iota(jnp.int32, d.shape, 1) + c * _BC1
    rarg = jnp.min(jnp.where(d == rmin, col, jnp.int32(2**30)),
                   axis=1, keepdims=True)

    @pl.when(c == 0)
    def _():
        minv[...] = jnp.full_like(minv[...], jnp.inf)
        mini[...] = jnp.zeros_like(mini[...])

    upd = rmin < minv[...]
    mini[...] = jnp.where(upd, rarg, mini[...])
    minv[...] = jnp.where(upd, rmin, minv[...])

    @pl.when(c == nc - 1)
    def _():
        idx_ref[...] = mini[...]


def _enc_body(idx_ref, w_ref, x_ref, enc_ref, q_ref, loss_ref, perp_ref,
              hist, sse, qacc):
    r = pl.program_id(0)
    nr = pl.num_programs(0)
    c = pl.program_id(1)
    nc = pl.num_programs(1)
    idx = idx_ref[...]
    col = jax.lax.broadcasted_iota(jnp.int32, (_BR2, _BC2), 1) + c * _BC2
    enc = (col == idx).astype(jnp.float32)
    enc_ref[...] = enc
    colsum = jnp.sum(enc, axis=0, keepdims=True)

    @pl.when(r == 0)
    def _():
        hist[:, pl.ds(c * _BC2, _BC2)] = colsum

    @pl.when(r > 0)
    def _():
        hist[:, pl.ds(c * _BC2, _BC2)] += colsum

    part = jax.lax.dot_general(enc, w_ref[...], (((1,), (0,)), ((), ())),
                               preferred_element_type=jnp.float32)

    @pl.when(c == 0)
    def _():
        qacc[...] = part

    @pl.when(c > 0)
    def _():
        qacc[...] += part

    @pl.when(c == nc - 1)
    def _():
        xt = x_ref[...]
        diff = qacc[...] - xt
        q_ref[...] = xt + diff
        tile_sse = jnp.sum(diff * diff)
        prev = jnp.where(r == 0, 0.0, sse[0, 0])
        sse[0, 0] = prev + tile_sse

    @pl.when((r == nr - 1) & (c == nc - 1))
    def _():
        loss_ref[0, 0] = sse[0, 0] * (1.25 / float(_N * _DIM))
        avg = hist[...] * (1.0 / float(_N))
        ent = jnp.sum(avg * jnp.log(avg + 1e-10))
        perp_ref[0, 0] = jnp.exp(-ent)


def kernel(inputs, w):
    x = jnp.transpose(inputs, (0, 2, 3, 1))
    input_shape = x.shape
    flat = x.reshape(_N, _DIM)

    dist, idx = pl.pallas_call(
        _dist_body,
        grid=(_N // _BR1, _NE // _BC1),
        in_specs=[
            pl.BlockSpec((_BR1, _DIM), lambda r, c: (r, 0)),
            pl.BlockSpec((_BC1, _DIM), lambda r, c: (c, 0)),
        ],
        out_specs=[
            pl.BlockSpec((_BR1, _BC1), lambda r, c: (r, c)),
            pl.BlockSpec((_BR1, 1), lambda r, c: (r, 0)),
        ],
        out_shape=[
            jax.ShapeDtypeStruct((_N, _NE), jnp.float32),
            jax.ShapeDtypeStruct((_N, 1), jnp.int32),
        ],
        scratch_shapes=[
            pltpu.VMEM((_BR1, 1), jnp.float32),
            pltpu.VMEM((_BR1, 1), jnp.int32),
            pltpu.VMEM((_BR1, 1), jnp.float32),
            pltpu.VMEM((1, _NE), jnp.float32),
            pltpu.VMEM((_NE, _DIM), jnp.float32),
        ],
    )(flat, w)

    enc, q, loss, perp = pl.pallas_call(
        _enc_body,
        grid=(_N // _BR2, _NE // _BC2),
        in_specs=[
            pl.BlockSpec((_BR2, 1), lambda r, c: (r, 0)),
            pl.BlockSpec((_BC2, _DIM), lambda r, c: (c, 0)),
            pl.BlockSpec((_BR2, _DIM), lambda r, c: (r, 0)),
        ],
        out_specs=[
            pl.BlockSpec((_BR2, _BC2), lambda r, c: (r, c)),
            pl.BlockSpec((_BR2, _DIM), lambda r, c: (r, 0)),
            pl.BlockSpec((1, 1), lambda r, c: (0, 0),
                         memory_space=pltpu.SMEM),
            pl.BlockSpec((1, 1), lambda r, c: (0, 0),
                         memory_space=pltpu.SMEM),
        ],
        out_shape=[
            jax.ShapeDtypeStruct((_N, _NE), jnp.float32),
            jax.ShapeDtypeStruct((_N, _DIM), jnp.float32),
            jax.ShapeDtypeStruct((1, 1), jnp.float32),
            jax.ShapeDtypeStruct((1, 1), jnp.float32),
        ],
        scratch_shapes=[
            pltpu.VMEM((1, _NE), jnp.float32),
            pltpu.SMEM((1, 1), jnp.float32),
            pltpu.VMEM((_BR2, _DIM), jnp.float32),
        ],
    )(idx, w, flat)

    quantized = jnp.transpose(q.reshape(input_shape), (0, 3, 1, 2))
    return (dist, quantized, loss[0, 0], enc, idx, perp[0, 0])


# speedup vs baseline: 8.8855x; 2.3779x over previous
import jax
import jax.numpy as jnp
from jax.experimental import pallas as pl
from jax.experimental.pallas import tpu as pltpu

_DIM = 32
_NE = 8192
_N = 8192
_BR1 = 256
_BR2 = 256


def _dist_body(x_ref, w_ref, d_ref, idx_ref, w2c, w2x):
    r = pl.program_id(0)

    @pl.when(r == 0)
    def _():
        wt = w_ref[...]
        w2x[...] = wt + wt
        w2c[...] = jnp.sum(wt * wt, axis=1)[None, :]

    x = x_ref[...]
    x2 = jnp.sum(x * x, axis=1, keepdims=True)
    mm2 = jax.lax.dot_general(x, w2x[...], (((1,), (1,)), ((), ())),
                              preferred_element_type=jnp.float32)
    d = (x2 - mm2) + w2c[...]
    d_ref[...] = d
    rmin = jnp.min(d, axis=1, keepdims=True)
    col = jax.lax.broadcasted_iota(jnp.int32, d.shape, 1)
    idx_ref[...] = jnp.min(jnp.where(d == rmin, col, jnp.int32(2**30)),
                           axis=1, keepdims=True)


def _enc_body(idx_ref, w_ref, x_ref, enc_ref, q_ref, loss_ref, perp_ref,
              hist, sse):
    r = pl.program_id(0)
    nr = pl.num_programs(0)
    idx = idx_ref[...]
    col = jax.lax.broadcasted_iota(jnp.int32, (_BR2, _NE), 1)
    enc = (col == idx).astype(jnp.float32)
    enc_ref[...] = enc
    colsum = jnp.sum(enc, axis=0, keepdims=True)

    @pl.when(r == 0)
    def _():
        hist[...] = colsum

    @pl.when(r > 0)
    def _():
        hist[...] += colsum

    q = jax.lax.dot_general(enc, w_ref[...], (((1,), (0,)), ((), ())),
                            preferred_element_type=jnp.float32)
    xt = x_ref[...]
    diff = q - xt
    q_ref[...] = xt + diff
    tile_sse = jnp.sum(diff * diff)
    prev = jnp.where(r == 0, 0.0, sse[0, 0])
    sse[0, 0] = prev + tile_sse

    @pl.when(r == nr - 1)
    def _():
        loss_ref[0, 0] = sse[0, 0] * (1.25 / float(_N * _DIM))
        avg = hist[...] * (1.0 / float(_N))
        ent = jnp.sum(avg * jnp.log(avg + 1e-10))
        perp_ref[0, 0] = jnp.exp(-ent)


def kernel(inputs, w):
    x = jnp.transpose(inputs, (0, 2, 3, 1))
    input_shape = x.shape
    flat = x.reshape(_N, _DIM)

    dist, idx = pl.pallas_call(
        _dist_body,
        grid=(_N // _BR1,),
        in_specs=[
            pl.BlockSpec((_BR1, _DIM), lambda r: (r, 0)),
            pl.BlockSpec((_NE, _DIM), lambda r: (0, 0)),
        ],
        out_specs=[
            pl.BlockSpec((_BR1, _NE), lambda r: (r, 0)),
            pl.BlockSpec((_BR1, 1), lambda r: (r, 0)),
        ],
        out_shape=[
            jax.ShapeDtypeStruct((_N, _NE), jnp.float32),
            jax.ShapeDtypeStruct((_N, 1), jnp.int32),
        ],
        scratch_shapes=[
            pltpu.VMEM((1, _NE), jnp.float32),
            pltpu.VMEM((_NE, _DIM), jnp.float32),
        ],
    )(flat, w)

    enc, q, loss, perp = pl.pallas_call(
        _enc_body,
        grid=(_N // _BR2,),
        in_specs=[
            pl.BlockSpec((_BR2, 1), lambda r: (r, 0)),
            pl.BlockSpec((_NE, _DIM), lambda r: (0, 0)),
            pl.BlockSpec((_BR2, _DIM), lambda r: (r, 0)),
        ],
        out_specs=[
            pl.BlockSpec((_BR2, _NE), lambda r: (r, 0)),
            pl.BlockSpec((_BR2, _DIM), lambda r: (r, 0)),
            pl.BlockSpec((1, 1), lambda r: (0, 0),
                         memory_space=pltpu.SMEM),
            pl.BlockSpec((1, 1), lambda r: (0, 0),
                         memory_space=pltpu.SMEM),
        ],
        out_shape=[
            jax.ShapeDtypeStruct((_N, _NE), jnp.float32),
            jax.ShapeDtypeStruct((_N, _DIM), jnp.float32),
            jax.ShapeDtypeStruct((1, 1), jnp.float32),
            jax.ShapeDtypeStruct((1, 1), jnp.float32),
        ],
        scratch_shapes=[
            pltpu.VMEM((1, _NE), jnp.float32),
            pltpu.SMEM((1, 1), jnp.float32),
        ],
    )(idx, w, flat)

    quantized = jnp.transpose(q.reshape(input_shape), (0, 3, 1, 2))
    return (dist, quantized, loss[0, 0], enc, idx, perp[0, 0])


# speedup vs baseline: 9.1768x; 1.0328x over previous
import jax
import jax.numpy as jnp
from jax.experimental import pallas as pl
from jax.experimental.pallas import tpu as pltpu

_DIM = 32
_NE = 8192
_N = 8192
_BR1 = 512
_BR2 = 512


def _dist_body(x_ref, w_ref, d_ref, idx_ref, w2c, w2x):
    r = pl.program_id(0)

    @pl.when(r == 0)
    def _():
        wt = w_ref[...]
        w2x[...] = wt + wt
        w2c[...] = jnp.sum(wt * wt, axis=1)[None, :]

    x = x_ref[...]
    x2 = jnp.sum(x * x, axis=1, keepdims=True)
    mm2 = jax.lax.dot_general(x, w2x[...], (((1,), (1,)), ((), ())),
                              preferred_element_type=jnp.float32)
    d = (x2 - mm2) + w2c[...]
    d_ref[...] = d
    rmin = jnp.min(d, axis=1, keepdims=True)
    col = jax.lax.broadcasted_iota(jnp.int32, d.shape, 1)
    idx_ref[...] = jnp.min(jnp.where(d == rmin, col, jnp.int32(2**30)),
                           axis=1, keepdims=True)


def _enc_body(idx_ref, w_ref, x_ref, enc_ref, q_ref, loss_ref, perp_ref,
              hist, sse):
    r = pl.program_id(0)
    nr = pl.num_programs(0)
    idx = idx_ref[...]
    col = jax.lax.broadcasted_iota(jnp.int32, (_BR2, _NE), 1)
    enc = (col == idx).astype(jnp.float32)
    enc_ref[...] = enc
    colsum = jnp.sum(enc, axis=0, keepdims=True)

    @pl.when(r == 0)
    def _():
        hist[...] = colsum

    @pl.when(r > 0)
    def _():
        hist[...] += colsum

    q = jax.lax.dot_general(enc, w_ref[...], (((1,), (0,)), ((), ())),
                            preferred_element_type=jnp.float32)
    xt = x_ref[...]
    diff = q - xt
    q_ref[...] = xt + diff
    tile_sse = jnp.sum(diff * diff)
    prev = jnp.where(r == 0, 0.0, sse[0, 0])
    sse[0, 0] = prev + tile_sse

    @pl.when(r == nr - 1)
    def _():
        loss_ref[0, 0] = sse[0, 0] * (1.25 / float(_N * _DIM))
        avg = hist[...] * (1.0 / float(_N))
        ent = jnp.sum(avg * jnp.log(avg + 1e-10))
        perp_ref[0, 0] = jnp.exp(-ent)


def kernel(inputs, w):
    x = jnp.transpose(inputs, (0, 2, 3, 1))
    input_shape = x.shape
    flat = x.reshape(_N, _DIM)

    dist, idx = pl.pallas_call(
        _dist_body,
        grid=(_N // _BR1,),
        in_specs=[
            pl.BlockSpec((_BR1, _DIM), lambda r: (r, 0)),
            pl.BlockSpec((_NE, _DIM), lambda r: (0, 0)),
        ],
        out_specs=[
            pl.BlockSpec((_BR1, _NE), lambda r: (r, 0)),
            pl.BlockSpec((_BR1, 1), lambda r: (r, 0)),
        ],
        out_shape=[
            jax.ShapeDtypeStruct((_N, _NE), jnp.float32),
            jax.ShapeDtypeStruct((_N, 1), jnp.int32),
        ],
        scratch_shapes=[
            pltpu.VMEM((1, _NE), jnp.float32),
            pltpu.VMEM((_NE, _DIM), jnp.float32),
        ],
    )(flat, w)

    enc, q, loss, perp = pl.pallas_call(
        _enc_body,
        grid=(_N // _BR2,),
        in_specs=[
            pl.BlockSpec((_BR2, 1), lambda r: (r, 0)),
            pl.BlockSpec((_NE, _DIM), lambda r: (0, 0)),
            pl.BlockSpec((_BR2, _DIM), lambda r: (r, 0)),
        ],
        out_specs=[
            pl.BlockSpec((_BR2, _NE), lambda r: (r, 0)),
            pl.BlockSpec((_BR2, _DIM), lambda r: (r, 0)),
            pl.BlockSpec((1, 1), lambda r: (0, 0),
                         memory_space=pltpu.SMEM),
            pl.BlockSpec((1, 1), lambda r: (0, 0),
                         memory_space=pltpu.SMEM),
        ],
        out_shape=[
            jax.ShapeDtypeStruct((_N, _NE), jnp.float32),
            jax.ShapeDtypeStruct((_N, _DIM), jnp.float32),
            jax.ShapeDtypeStruct((1, 1), jnp.float32),
            jax.ShapeDtypeStruct((1, 1), jnp.float32),
        ],
        scratch_shapes=[
            pltpu.VMEM((1, _NE), jnp.float32),
            pltpu.SMEM((1, 1), jnp.float32),
        ],
    )(idx, w, flat)

    quantized = jnp.transpose(q.reshape(input_shape), (0, 3, 1, 2))
    return (dist, quantized, loss[0, 0], enc, idx, perp[0, 0])
